# baseline (speedup 1.0000x reference)
import functools

import jax
import jax.numpy as jnp
from jax import lax
from jax.experimental import pallas as pl
from jax.experimental.pallas import tpu as pltpu
from jax.experimental.pallas import tpu_sc as plsc

B, N, H, E, D_EDGE, NUM_SPATIAL = 16, 256, 32, 131072, 16, 512
NSLOT = 4096
LANES = 16
EDGE_CHUNK = 2048


def _edge_select_kernel(ei0_hbm, ei1_hbm, ei2_hbm, ea_hbm, sel_hbm, w_hbm, r0,
                        r1, r2, m2d, w_v, kbuf, idx8_v, gbuf, selflat, sem):
  wid = lax.axis_index("s") * 2 + lax.axis_index("c")

  @pl.when(wid == 0)
  def _():
    iota = lax.iota(jnp.int32, LANES)
    zeros_i = jnp.zeros((LANES,), jnp.int32)
    zeros_f = jnp.zeros((LANES,), jnp.float32)
    ones_f = jnp.ones((LANES,), jnp.float32)

    def init_body(k, _):
      w_v[pl.ds(k * LANES, LANES)] = zeros_f
      row = k >> 3
      col = (k & 7) * LANES
      m2d[row, pl.ds(col, LANES)] = zeros_i
      return 0

    lax.fori_loop(0, NSLOT // LANES, init_body, 0)
    kbuf[pl.ds(LANES, LANES)] = jnp.full((LANES,), 2**30, jnp.int32)

    nsteps = EDGE_CHUNK // LANES

    def issue_chunk(c, par):
      o = par * EDGE_CHUNK
      pltpu.async_copy(ei0_hbm.at[pl.ds(c * EDGE_CHUNK, EDGE_CHUNK)],
                       r0.at[pl.ds(o, EDGE_CHUNK)], sem)
      pltpu.async_copy(ei1_hbm.at[pl.ds(c * EDGE_CHUNK, EDGE_CHUNK)],
                       r1.at[pl.ds(o, EDGE_CHUNK)], sem)
      pltpu.async_copy(ei2_hbm.at[pl.ds(c * EDGE_CHUNK, EDGE_CHUNK)],
                       r2.at[pl.ds(o, EDGE_CHUNK)], sem)

    def drain_chunk():
      for _ in range(3):
        pltpu.make_async_copy(ei0_hbm.at[pl.ds(0, EDGE_CHUNK)],
                              r0.at[pl.ds(0, EDGE_CHUNK)], sem).wait()

    issue_chunk(0, 0)

    def chunk_body(c, _):
      par = jnp.bitwise_and(c, 1)
      drain_chunk()

      @pl.when(c + 1 < E // EDGE_CHUNK)
      def _():
        issue_chunk(c + 1, jnp.bitwise_xor(par, 1))

      def step_body(s, _):
        off = par * EDGE_CHUNK + s * LANES
        ia = r0[pl.ds(off, LANES)]
        ib = r1[pl.ds(off, LANES)]
        ic = r2[pl.ds(off, LANES)]
        slot = ia * 256 + ib * 16 + ic
        e = (c * EDGE_CHUNK + s * LANES) + iota
        key = slot * (2**17) + e
        sk, _unused = plsc.sort_key_val(key, e)
        kbuf[pl.ds(0, LANES)] = sk
        nxt = plsc.load_gather(kbuf, [iota + 1])
        slot_s = lax.shift_right_arithmetic(sk, 17)
        keep = jnp.not_equal(slot_s, lax.shift_right_arithmetic(nxt, 17))
        es = jnp.bitwise_and(sk, 2**17 - 1)
        row = lax.shift_right_arithmetic(slot_s, 7)
        col = jnp.bitwise_and(slot_s, 127)
        plsc.store_scatter(m2d, [row, col], es, mask=keep)
        plsc.store_scatter(w_v, [slot_s], ones_f, mask=keep)
        return 0

      lax.fori_loop(0, nsteps, step_body, 0)
      return 0

    lax.fori_loop(0, E // EDGE_CHUNK, chunk_body, 0)

    def chunk_gather(r, _):

      def idx_body(g, _):
        mv = plsc.load_gather(m2d, [zeros_i + r, g * LANES + iota])
        idx8_v[pl.ds(g * LANES, LANES)] = lax.shift_right_arithmetic(mv, 3)
        return 0

      lax.fori_loop(0, 8, idx_body, 0)
      pltpu.async_copy(ea_hbm.at[idx8_v], gbuf, sem).wait()

      def ext_body(k, _):
        mkv = plsc.load_gather(m2d, [zeros_i + r, zeros_i + k])
        sub = jnp.bitwise_and(mkv, 7) * D_EDGE
        val = plsc.load_gather(gbuf, [zeros_i + k, sub + iota])
        selflat[pl.ds(k * D_EDGE, LANES)] = val
        return 0

      lax.fori_loop(0, 128, ext_body, 0)
      pltpu.sync_copy(selflat,
                      sel_hbm.at[pl.ds(r * 128 * D_EDGE, 128 * D_EDGE)])
      return 0

    lax.fori_loop(0, NSLOT // 128, chunk_gather, 0)
    pltpu.sync_copy(w_v, w_hbm)


def _edge_select(edge_index, edge_attr):
  mesh = plsc.VectorSubcoreMesh(core_axis_name="c", subcore_axis_name="s")
  f = pl.kernel(
      _edge_select_kernel,
      out_type=(jax.ShapeDtypeStruct((NSLOT * D_EDGE,), jnp.float32),
                jax.ShapeDtypeStruct((NSLOT,), jnp.float32)),
      mesh=mesh,
      scratch_types=[
          pltpu.VMEM((2 * EDGE_CHUNK,), jnp.int32),
          pltpu.VMEM((2 * EDGE_CHUNK,), jnp.int32),
          pltpu.VMEM((2 * EDGE_CHUNK,), jnp.int32),
          pltpu.VMEM((32, 128), jnp.int32),
          pltpu.VMEM((NSLOT,), jnp.float32),
          pltpu.VMEM((2 * LANES,), jnp.int32),
          pltpu.VMEM((128,), jnp.int32),
          pltpu.VMEM((128, 128), jnp.float32),
          pltpu.VMEM((128 * D_EDGE,), jnp.float32),
          pltpu.SemaphoreType.DMA,
      ],
      compiler_params=pltpu.CompilerParams(needs_layout_passes=False),
  )
  return f(edge_index[0], edge_index[1], edge_index[2],
           edge_attr.reshape(E // 8, 128))


NP = N + 1
ABP = 264
OROW = NP
OBUF = H * OROW


ABV = 272


def _assemble_kernel(ab_hbm, sp_hbm, emb_hbm, wedge_hbm, virt_hbm, sel_hbm,
                     wm_hbm, out_hbm, ab_v, sp_v, sp32, tvec, attr_v, attr_t,
                     outf, emb_v, wedge_v, virt_v, wm_v, sem_in, sem_out):
  wid = lax.axis_index("s") * 2 + lax.axis_index("c")
  iota = lax.iota(jnp.int32, LANES)
  zeros_i = jnp.zeros((LANES,), jnp.int32)

  pltpu.sync_copy(emb_hbm, emb_v)
  pltpu.sync_copy(wedge_hbm, wedge_v)
  pltpu.sync_copy(virt_hbm, virt_v)
  pltpu.sync_copy(wm_hbm, wm_v)

  ntask = jnp.where(wid < LANES, 129, 128)

  def advance(b, i_out):
    i2 = i_out + 32
    wrap = i2 >= NP
    return jnp.where(wrap, b + 1, b), jnp.where(wrap, i2 - NP, i2)

  def issue_inputs(b, i_out, par):
    icl = jnp.maximum(i_out - 1, 0)
    sbase = (b * 16 + jnp.minimum(icl, 15)) * 16
    pltpu.async_copy(ab_hbm.at[b, i_out], ab_v.at[pl.ds(par * ABV, ABP)],
                     sem_in)
    pltpu.async_copy(sp_hbm.at[b, icl], sp_v.at[pl.ds(par * N, N)], sem_in)
    pltpu.async_copy(sel_hbm.at[pl.ds(sbase * D_EDGE, 256)],
                     attr_v.at[pl.ds(par * 256, 256)], sem_in)

  def drain_inputs():
    pltpu.make_async_copy(ab_hbm.at[0, 0], ab_v.at[pl.ds(0, ABP)],
                          sem_in).wait()
    pltpu.make_async_copy(sp_hbm.at[0, 0], sp_v.at[pl.ds(0, N)],
                          sem_in).wait()
    pltpu.make_async_copy(sel_hbm.at[pl.ds(0, 256)], attr_v.at[pl.ds(0, 256)],
                          sem_in).wait()

  def drain_outputs():
    pltpu.make_async_copy(out_hbm.at[0, 0], outf.at[pl.ds(0, OBUF)],
                          sem_out).wait()

  issue_inputs(jnp.int32(0), jnp.int32(wid), 0)

  def task_body(k, carry):
    b, i_out = carry
    par = jnp.bitwise_and(k, 1)
    pab = par * ABV
    psp = par * N
    pat = par * 256
    pob = par * OBUF
    is_row0 = i_out == 0
    icl = jnp.maximum(i_out - 1, 0)
    is_edge = jnp.logical_and(jnp.logical_not(is_row0), icl < 16)
    sbase = (b * 16 + jnp.minimum(icl, 15)) * 16

    drain_inputs()
    bn, in_ = advance(b, i_out)

    @pl.when(k + 1 < ntask)
    def _():
      issue_inputs(bn, in_, jnp.bitwise_xor(par, 1))

    @pl.when(k >= 2)
    def _():
      drain_outputs()

    for jg in range(16):
      o = jg * LANES
      tvec[pl.ds(o, LANES)] = 2.0 * plsc.load_gather(ab_v,
                                                     [iota + (pab + o + 1)])
      sp32[pl.ds(o, LANES)] = sp_v[pl.ds(psp + o, LANES)] * 32

    ab0 = ab_v[pl.ds(pab, LANES)][0]

    for hg in range(2):
      hv = iota + hg * LANES
      val = 2.0 * ab0 + virt_v[pl.ds(hg * LANES, LANES)]
      plsc.store_scatter(outf, [pob + hv * OROW], val)

    def jg_body(jg, _):
      o = jg * LANES
      sp32g = sp32[pl.ds(o, LANES)]
      tv = tvec[pl.ds(o, LANES)]
      basec = (pob + o + 1) + iota
      for h in range(H):
        g = plsc.load_gather(emb_v, [sp32g + h])
        plsc.store_scatter(outf, [basec + h * OROW], tv + g)
      return 0

    lax.fori_loop(0, 16, jg_body, 0)

    @pl.when(is_row0)
    def _():

      def vh_body(h, _):
        vhv = plsc.load_gather(virt_v, [zeros_i + h])
        base = (pob + h * OROW + 1) + iota
        for jg in range(16):
          o = jg * LANES
          plsc.store_scatter(outf, [base + o], tvec[pl.ds(o, LANES)] + vhv)
        return 0

      lax.fori_loop(0, H, vh_body, 0)

    @pl.when(is_edge)
    def _():
      wmask = wm_v[pl.ds(sbase, LANES)]
      for d in range(D_EDGE):
        attr_t[pl.ds(d * LANES, LANES)] = plsc.load_gather(
            attr_v, [pat + iota * D_EDGE + d])

      def eh_body(h, _):
        acc = jnp.zeros((LANES,), jnp.float32)
        for d in range(D_EDGE):
          wv = plsc.load_gather(wedge_v, [zeros_i + (d * H + h)])
          acc = acc + attr_t[pl.ds(d * LANES, LANES)] * wv
        idxv = (pob + h * OROW + 1) + iota
        cur = plsc.load_gather(outf, [idxv])
        plsc.store_scatter(outf, [idxv], cur + acc * wmask)
        return 0

      lax.fori_loop(0, H, eh_body, 0)

    pltpu.async_copy(outf.at[pl.ds(pob, OBUF)], out_hbm.at[b, i_out],
                     sem_out)

    return (bn, in_)

  lax.fori_loop(0, ntask, task_body, (jnp.int32(0), jnp.int32(wid)))
  drain_outputs()
  drain_outputs()


def _assemble(ab_pad, spatial_pos, emb_flat, wedge_flat, virt, sel_flat,
              w_mask):
  mesh = plsc.VectorSubcoreMesh(core_axis_name="c", subcore_axis_name="s")
  f = pl.kernel(
      _assemble_kernel,
      out_type=jax.ShapeDtypeStruct((B, NP, H * NP), jnp.float32),
      mesh=mesh,
      scratch_types=[
          pltpu.VMEM((2 * ABV,), jnp.float32),
          pltpu.VMEM((2 * N,), jnp.int32),
          pltpu.VMEM((N,), jnp.int32),
          pltpu.VMEM((N,), jnp.float32),
          pltpu.VMEM((2 * 256,), jnp.float32),
          pltpu.VMEM((256,), jnp.float32),
          pltpu.VMEM((2 * OBUF,), jnp.float32),
          pltpu.VMEM((NUM_SPATIAL * H,), jnp.float32),
          pltpu.VMEM((D_EDGE * H,), jnp.float32),
          pltpu.VMEM((H,), jnp.float32),
          pltpu.VMEM((NSLOT,), jnp.float32),
          pltpu.SemaphoreType.DMA,
          pltpu.SemaphoreType.DMA,
      ],
      compiler_params=pltpu.CompilerParams(needs_layout_passes=False,
                                           use_tc_tiling_on_sc=False),
  )
  return f(ab_pad, spatial_pos, emb_flat, wedge_flat, virt, sel_flat, w_mask)


def kernel(x, edge_index, edge_attr, path_index, attn_bias, spatial_pos,
           W_edge, spatial_emb, virt_dist):
  del x, path_index
  ab_pad = jnp.pad(attn_bias, ((0, 0), (0, 0), (0, ABP - NP)))
  sel, w_mask = _edge_select(edge_index, edge_attr)
  out = _assemble(ab_pad, spatial_pos, spatial_emb.reshape(-1),
                  W_edge.reshape(-1), virt_dist.reshape(-1), sel, w_mask)
  return out.reshape(B, NP, H, NP).transpose(0, 2, 1, 3)


# speedup vs baseline: 1.3547x; 1.3547x over previous
import functools

import jax
import jax.numpy as jnp
from jax import lax
from jax.experimental import pallas as pl
from jax.experimental.pallas import tpu as pltpu
from jax.experimental.pallas import tpu_sc as plsc

B, N, H, E, D_EDGE, NUM_SPATIAL = 16, 256, 32, 131072, 16, 512
NSLOT = 4096
LANES = 16
EDGE_CHUNK = 2048


def _edge_select_kernel(ei0_hbm, ei1_hbm, ei2_hbm, ea_hbm, sel_hbm, w_hbm, r0,
                        r1, r2, m2d, w_v, kbuf, idx8_v, gbuf, selflat, sem):
  wid = lax.axis_index("s") * 2 + lax.axis_index("c")

  @pl.when(wid == 0)
  def _():
    iota = lax.iota(jnp.int32, LANES)
    zeros_i = jnp.zeros((LANES,), jnp.int32)
    zeros_f = jnp.zeros((LANES,), jnp.float32)
    ones_f = jnp.ones((LANES,), jnp.float32)

    def init_body(k, _):
      w_v[pl.ds(k * LANES, LANES)] = zeros_f
      row = k >> 3
      col = (k & 7) * LANES
      m2d[row, pl.ds(col, LANES)] = zeros_i
      return 0

    lax.fori_loop(0, NSLOT // LANES, init_body, 0)
    kbuf[pl.ds(LANES, LANES)] = jnp.full((LANES,), 2**30, jnp.int32)

    nsteps = EDGE_CHUNK // LANES

    def issue_chunk(c, par):
      o = par * EDGE_CHUNK
      pltpu.async_copy(ei0_hbm.at[pl.ds(c * EDGE_CHUNK, EDGE_CHUNK)],
                       r0.at[pl.ds(o, EDGE_CHUNK)], sem)
      pltpu.async_copy(ei1_hbm.at[pl.ds(c * EDGE_CHUNK, EDGE_CHUNK)],
                       r1.at[pl.ds(o, EDGE_CHUNK)], sem)
      pltpu.async_copy(ei2_hbm.at[pl.ds(c * EDGE_CHUNK, EDGE_CHUNK)],
                       r2.at[pl.ds(o, EDGE_CHUNK)], sem)

    def drain_chunk():
      for _ in range(3):
        pltpu.make_async_copy(ei0_hbm.at[pl.ds(0, EDGE_CHUNK)],
                              r0.at[pl.ds(0, EDGE_CHUNK)], sem).wait()

    issue_chunk(0, 0)

    def chunk_body(c, _):
      par = jnp.bitwise_and(c, 1)
      drain_chunk()

      @pl.when(c + 1 < E // EDGE_CHUNK)
      def _():
        issue_chunk(c + 1, jnp.bitwise_xor(par, 1))

      def step_body(s, _):
        off = par * EDGE_CHUNK + s * LANES
        ia = r0[pl.ds(off, LANES)]
        ib = r1[pl.ds(off, LANES)]
        ic = r2[pl.ds(off, LANES)]
        slot = ia * 256 + ib * 16 + ic
        e = (c * EDGE_CHUNK + s * LANES) + iota
        key = slot * (2**17) + e
        sk, _unused = plsc.sort_key_val(key, e)
        kbuf[pl.ds(0, LANES)] = sk
        nxt = plsc.load_gather(kbuf, [iota + 1])
        slot_s = lax.shift_right_arithmetic(sk, 17)
        keep = jnp.not_equal(slot_s, lax.shift_right_arithmetic(nxt, 17))
        es = jnp.bitwise_and(sk, 2**17 - 1)
        row = lax.shift_right_arithmetic(slot_s, 7)
        col = jnp.bitwise_and(slot_s, 127)
        plsc.store_scatter(m2d, [row, col], es, mask=keep)
        plsc.store_scatter(w_v, [slot_s], ones_f, mask=keep)
        return 0

      lax.fori_loop(0, nsteps, step_body, 0)
      return 0

    lax.fori_loop(0, E // EDGE_CHUNK, chunk_body, 0)

    def chunk_gather(r, _):

      def idx_body(g, _):
        mv = plsc.load_gather(m2d, [zeros_i + r, g * LANES + iota])
        idx8_v[pl.ds(g * LANES, LANES)] = lax.shift_right_arithmetic(mv, 3)
        return 0

      lax.fori_loop(0, 8, idx_body, 0)
      pltpu.async_copy(ea_hbm.at[idx8_v], gbuf, sem).wait()

      def ext_body(k, _):
        mkv = plsc.load_gather(m2d, [zeros_i + r, zeros_i + k])
        sub = jnp.bitwise_and(mkv, 7) * D_EDGE
        val = plsc.load_gather(gbuf, [zeros_i + k, sub + iota])
        selflat[pl.ds(k * D_EDGE, LANES)] = val
        return 0

      lax.fori_loop(0, 128, ext_body, 0)
      pltpu.sync_copy(selflat,
                      sel_hbm.at[pl.ds(r * 128 * D_EDGE, 128 * D_EDGE)])
      return 0

    lax.fori_loop(0, NSLOT // 128, chunk_gather, 0)
    pltpu.sync_copy(w_v, w_hbm)


def _edge_select(edge_index, edge_attr):
  mesh = plsc.VectorSubcoreMesh(core_axis_name="c", subcore_axis_name="s")
  f = pl.kernel(
      _edge_select_kernel,
      out_type=(jax.ShapeDtypeStruct((NSLOT * D_EDGE,), jnp.float32),
                jax.ShapeDtypeStruct((NSLOT,), jnp.float32)),
      mesh=mesh,
      scratch_types=[
          pltpu.VMEM((2 * EDGE_CHUNK,), jnp.int32),
          pltpu.VMEM((2 * EDGE_CHUNK,), jnp.int32),
          pltpu.VMEM((2 * EDGE_CHUNK,), jnp.int32),
          pltpu.VMEM((32, 128), jnp.int32),
          pltpu.VMEM((NSLOT,), jnp.float32),
          pltpu.VMEM((2 * LANES,), jnp.int32),
          pltpu.VMEM((128,), jnp.int32),
          pltpu.VMEM((128, 128), jnp.float32),
          pltpu.VMEM((128 * D_EDGE,), jnp.float32),
          pltpu.SemaphoreType.DMA,
      ],
      compiler_params=pltpu.CompilerParams(needs_layout_passes=False),
  )
  return f(edge_index[0], edge_index[1], edge_index[2],
           edge_attr.reshape(E // 8, 128))


NP = N + 1
ABP = 264
OROW = 272
OBUF = H * OROW


ABV = 272


def _assemble_kernel(ab_hbm, sp_hbm, emb_hbm, wedge_hbm, virt_hbm, sel_hbm,
                     wm_hbm, out_hbm, ab_v, sp_v, sp32, tvec, attr_v, attr_t,
                     outf, emb_v, wedge_v, virt_v, wm_v, sem_in, sem_out):
  wid = lax.axis_index("s") * 2 + lax.axis_index("c")
  iota = lax.iota(jnp.int32, LANES)
  zeros_i = jnp.zeros((LANES,), jnp.int32)

  pltpu.sync_copy(emb_hbm, emb_v)
  pltpu.sync_copy(wedge_hbm, wedge_v)
  pltpu.sync_copy(virt_hbm, virt_v)
  pltpu.sync_copy(wm_hbm, wm_v)

  ntask = jnp.where(wid < LANES, 129, 128)

  def advance(b, i_out):
    i2 = i_out + 32
    wrap = i2 >= NP
    return jnp.where(wrap, b + 1, b), jnp.where(wrap, i2 - NP, i2)

  def issue_inputs(b, i_out, par):
    icl = jnp.maximum(i_out - 1, 0)
    sbase = (b * 16 + jnp.minimum(icl, 15)) * 16
    pltpu.async_copy(ab_hbm.at[b, i_out], ab_v.at[pl.ds(par * ABV, ABP)],
                     sem_in)
    pltpu.async_copy(sp_hbm.at[b, icl], sp_v.at[pl.ds(par * N, N)], sem_in)
    pltpu.async_copy(sel_hbm.at[pl.ds(sbase * D_EDGE, 256)],
                     attr_v.at[pl.ds(par * 256, 256)], sem_in)

  def drain_inputs():
    pltpu.make_async_copy(ab_hbm.at[0, 0], ab_v.at[pl.ds(0, ABP)],
                          sem_in).wait()
    pltpu.make_async_copy(sp_hbm.at[0, 0], sp_v.at[pl.ds(0, N)],
                          sem_in).wait()
    pltpu.make_async_copy(sel_hbm.at[pl.ds(0, 256)], attr_v.at[pl.ds(0, 256)],
                          sem_in).wait()

  def drain_outputs():
    for _ in range(H):
      pltpu.make_async_copy(out_hbm.at[0, 0, 0], outf.at[pl.ds(0, NP)],
                            sem_out).wait()

  issue_inputs(jnp.int32(0), jnp.int32(wid), 0)

  def task_body(k, carry):
    b, i_out = carry
    par = jnp.bitwise_and(k, 1)
    pab = par * ABV
    psp = par * N
    pat = par * 256
    pob = par * OBUF
    is_row0 = i_out == 0
    icl = jnp.maximum(i_out - 1, 0)
    is_edge = jnp.logical_and(jnp.logical_not(is_row0), icl < 16)
    sbase = (b * 16 + jnp.minimum(icl, 15)) * 16

    drain_inputs()
    bn, in_ = advance(b, i_out)

    @pl.when(k + 1 < ntask)
    def _():
      issue_inputs(bn, in_, jnp.bitwise_xor(par, 1))

    @pl.when(k >= 2)
    def _():
      drain_outputs()

    for jg in range(16):
      o = jg * LANES
      tvec[pl.ds(o, LANES)] = 2.0 * plsc.load_gather(ab_v,
                                                     [iota + (pab + o + 1)])
      sp32[pl.ds(o, LANES)] = sp_v[pl.ds(psp + o, LANES)] * 32

    ab0 = ab_v[pl.ds(pab, LANES)][0]

    for hg in range(2):
      hv = iota + hg * LANES
      val = 2.0 * ab0 + virt_v[pl.ds(hg * LANES, LANES)]
      plsc.store_scatter(outf, [pob + hv * OROW], val)

    def jg_body(jg, _):
      o = jg * LANES
      sp32g = sp32[pl.ds(o, LANES)]
      tv = tvec[pl.ds(o, LANES)]
      base = pob + o + 1
      for h in range(H):
        g = plsc.load_gather(emb_v, [sp32g + h])
        outf[pl.ds(base + h * OROW, LANES)] = tv + g
      return 0

    lax.fori_loop(0, 16, jg_body, 0)

    @pl.when(is_row0)
    def _():

      def vh_body(h, _):
        vhv = plsc.load_gather(virt_v, [zeros_i + h])
        base = (pob + h * OROW + 1) + iota
        for jg in range(16):
          o = jg * LANES
          plsc.store_scatter(outf, [base + o], tvec[pl.ds(o, LANES)] + vhv)
        return 0

      lax.fori_loop(0, H, vh_body, 0)

    @pl.when(is_edge)
    def _():
      wmask = wm_v[pl.ds(sbase, LANES)]
      for d in range(D_EDGE):
        attr_t[pl.ds(d * LANES, LANES)] = plsc.load_gather(
            attr_v, [pat + iota * D_EDGE + d])

      def eh_body(h, _):
        acc = jnp.zeros((LANES,), jnp.float32)
        for d in range(D_EDGE):
          wv = plsc.load_gather(wedge_v, [zeros_i + (d * H + h)])
          acc = acc + attr_t[pl.ds(d * LANES, LANES)] * wv
        idxv = (pob + h * OROW + 1) + iota
        cur = plsc.load_gather(outf, [idxv])
        plsc.store_scatter(outf, [idxv], cur + acc * wmask)
        return 0

      lax.fori_loop(0, H, eh_body, 0)

    for h in range(H):
      pltpu.async_copy(outf.at[pl.ds(pob + h * OROW, NP)],
                       out_hbm.at[b, h, i_out], sem_out)

    return (bn, in_)

  lax.fori_loop(0, ntask, task_body, (jnp.int32(0), jnp.int32(wid)))
  drain_outputs()
  drain_outputs()


def _assemble(ab_pad, spatial_pos, emb_flat, wedge_flat, virt, sel_flat,
              w_mask):
  mesh = plsc.VectorSubcoreMesh(core_axis_name="c", subcore_axis_name="s")
  f = pl.kernel(
      _assemble_kernel,
      out_type=jax.ShapeDtypeStruct((B, H, NP, NP), jnp.float32),
      mesh=mesh,
      scratch_types=[
          pltpu.VMEM((2 * ABV,), jnp.float32),
          pltpu.VMEM((2 * N,), jnp.int32),
          pltpu.VMEM((N,), jnp.int32),
          pltpu.VMEM((N,), jnp.float32),
          pltpu.VMEM((2 * 256,), jnp.float32),
          pltpu.VMEM((256,), jnp.float32),
          pltpu.VMEM((2 * OBUF,), jnp.float32),
          pltpu.VMEM((NUM_SPATIAL * H,), jnp.float32),
          pltpu.VMEM((D_EDGE * H,), jnp.float32),
          pltpu.VMEM((H,), jnp.float32),
          pltpu.VMEM((NSLOT,), jnp.float32),
          pltpu.SemaphoreType.DMA,
          pltpu.SemaphoreType.DMA,
      ],
      compiler_params=pltpu.CompilerParams(needs_layout_passes=False,
                                           use_tc_tiling_on_sc=False),
  )
  return f(ab_pad, spatial_pos, emb_flat, wedge_flat, virt, sel_flat, w_mask)


def kernel(x, edge_index, edge_attr, path_index, attn_bias, spatial_pos,
           W_edge, spatial_emb, virt_dist):
  del x, path_index
  ab_pad = jnp.pad(attn_bias, ((0, 0), (0, 0), (0, ABP - NP)))
  sel, w_mask = _edge_select(edge_index, edge_attr)
  out = _assemble(ab_pad, spatial_pos, spatial_emb.reshape(-1),
                  W_edge.reshape(-1), virt_dist.reshape(-1), sel, w_mask)
  return out


# speedup vs baseline: 1.4511x; 1.0712x over previous
import functools

import jax
import jax.numpy as jnp
from jax import lax
from jax.experimental import pallas as pl
from jax.experimental.pallas import tpu as pltpu
from jax.experimental.pallas import tpu_sc as plsc

B, N, H, E, D_EDGE, NUM_SPATIAL = 16, 256, 32, 131072, 16, 512
NSLOT = 4096
LANES = 16
EDGE_CHUNK = 2048


def _edge_select_kernel(ei0_hbm, ei1_hbm, ei2_hbm, ea_hbm, sel_hbm, w_hbm, r0,
                        r1, r2, m2d, w_v, kbuf, idx8_v, gbuf, selflat, sem):
  wid = lax.axis_index("s") * 2 + lax.axis_index("c")

  @pl.when(wid == 0)
  def _():
    iota = lax.iota(jnp.int32, LANES)
    zeros_i = jnp.zeros((LANES,), jnp.int32)
    zeros_f = jnp.zeros((LANES,), jnp.float32)
    ones_f = jnp.ones((LANES,), jnp.float32)

    def init_body(k, _):
      w_v[pl.ds(k * LANES, LANES)] = zeros_f
      row = k >> 3
      col = (k & 7) * LANES
      m2d[row, pl.ds(col, LANES)] = zeros_i
      return 0

    lax.fori_loop(0, NSLOT // LANES, init_body, 0)
    kbuf[pl.ds(LANES, LANES)] = jnp.full((LANES,), 2**30, jnp.int32)

    nsteps = EDGE_CHUNK // LANES

    def issue_chunk(c, par):
      o = par * EDGE_CHUNK
      pltpu.async_copy(ei0_hbm.at[pl.ds(c * EDGE_CHUNK, EDGE_CHUNK)],
                       r0.at[pl.ds(o, EDGE_CHUNK)], sem)
      pltpu.async_copy(ei1_hbm.at[pl.ds(c * EDGE_CHUNK, EDGE_CHUNK)],
                       r1.at[pl.ds(o, EDGE_CHUNK)], sem)
      pltpu.async_copy(ei2_hbm.at[pl.ds(c * EDGE_CHUNK, EDGE_CHUNK)],
                       r2.at[pl.ds(o, EDGE_CHUNK)], sem)

    def drain_chunk():
      for _ in range(3):
        pltpu.make_async_copy(ei0_hbm.at[pl.ds(0, EDGE_CHUNK)],
                              r0.at[pl.ds(0, EDGE_CHUNK)], sem).wait()

    issue_chunk(0, 0)

    def chunk_body(c, _):
      par = jnp.bitwise_and(c, 1)
      drain_chunk()

      @pl.when(c + 1 < E // EDGE_CHUNK)
      def _():
        issue_chunk(c + 1, jnp.bitwise_xor(par, 1))

      def step_body(s, _):
        off = par * EDGE_CHUNK + s * LANES
        ia = r0[pl.ds(off, LANES)]
        ib = r1[pl.ds(off, LANES)]
        ic = r2[pl.ds(off, LANES)]
        slot = ia * 256 + ib * 16 + ic
        e = (c * EDGE_CHUNK + s * LANES) + iota
        key = slot * (2**17) + e
        sk, _unused = plsc.sort_key_val(key, e)
        kbuf[pl.ds(0, LANES)] = sk
        nxt = plsc.load_gather(kbuf, [iota + 1])
        slot_s = lax.shift_right_arithmetic(sk, 17)
        keep = jnp.not_equal(slot_s, lax.shift_right_arithmetic(nxt, 17))
        es = jnp.bitwise_and(sk, 2**17 - 1)
        row = lax.shift_right_arithmetic(slot_s, 7)
        col = jnp.bitwise_and(slot_s, 127)
        plsc.store_scatter(m2d, [row, col], es, mask=keep)
        plsc.store_scatter(w_v, [slot_s], ones_f, mask=keep)
        return 0

      lax.fori_loop(0, nsteps, step_body, 0)
      return 0

    lax.fori_loop(0, E // EDGE_CHUNK, chunk_body, 0)

    def chunk_gather(r, _):

      def idx_body(g, _):
        mv = plsc.load_gather(m2d, [zeros_i + r, g * LANES + iota])
        idx8_v[pl.ds(g * LANES, LANES)] = lax.shift_right_arithmetic(mv, 3)
        return 0

      lax.fori_loop(0, 8, idx_body, 0)
      pltpu.async_copy(ea_hbm.at[idx8_v], gbuf, sem).wait()

      def ext_body(k, _):
        mkv = plsc.load_gather(m2d, [zeros_i + r, zeros_i + k])
        sub = jnp.bitwise_and(mkv, 7) * D_EDGE
        val = plsc.load_gather(gbuf, [zeros_i + k, sub + iota])
        selflat[pl.ds(k * D_EDGE, LANES)] = val
        return 0

      lax.fori_loop(0, 128, ext_body, 0)
      pltpu.sync_copy(selflat,
                      sel_hbm.at[pl.ds(r * 128 * D_EDGE, 128 * D_EDGE)])
      return 0

    lax.fori_loop(0, NSLOT // 128, chunk_gather, 0)
    pltpu.sync_copy(w_v, w_hbm)


def _edge_select(edge_index, edge_attr):
  mesh = plsc.VectorSubcoreMesh(core_axis_name="c", subcore_axis_name="s")
  f = pl.kernel(
      _edge_select_kernel,
      out_type=(jax.ShapeDtypeStruct((NSLOT * D_EDGE,), jnp.float32),
                jax.ShapeDtypeStruct((NSLOT,), jnp.float32)),
      mesh=mesh,
      scratch_types=[
          pltpu.VMEM((2 * EDGE_CHUNK,), jnp.int32),
          pltpu.VMEM((2 * EDGE_CHUNK,), jnp.int32),
          pltpu.VMEM((2 * EDGE_CHUNK,), jnp.int32),
          pltpu.VMEM((32, 128), jnp.int32),
          pltpu.VMEM((NSLOT,), jnp.float32),
          pltpu.VMEM((2 * LANES,), jnp.int32),
          pltpu.VMEM((128,), jnp.int32),
          pltpu.VMEM((128, 128), jnp.float32),
          pltpu.VMEM((128 * D_EDGE,), jnp.float32),
          pltpu.SemaphoreType.DMA,
      ],
      compiler_params=pltpu.CompilerParams(needs_layout_passes=False),
  )
  return f(edge_index[0], edge_index[1], edge_index[2],
           edge_attr.reshape(E // 8, 128))


NP = N + 1
ABP = 264
OROW = 272
OBUF = H * OROW


ABV = 272


def _assemble_kernel(ab_hbm, sp_hbm, emb_hbm, wedge_hbm, virt_hbm, sel_hbm,
                     wm_hbm, out_hbm, ab_v, sp_v, sp32, tvec, attr_v, attr_t,
                     outf, emb_v, wedge_v, virt_v, wm_v, sem_in, sem_out):
  wid = lax.axis_index("s") * 2 + lax.axis_index("c")
  iota = lax.iota(jnp.int32, LANES)
  zeros_i = jnp.zeros((LANES,), jnp.int32)

  pltpu.sync_copy(emb_hbm, emb_v)
  pltpu.sync_copy(wedge_hbm, wedge_v)
  pltpu.sync_copy(virt_hbm, virt_v)
  pltpu.sync_copy(wm_hbm, wm_v)

  ntask = jnp.where(wid < LANES, 129, 128)

  def advance(b, i_out):
    i2 = i_out + 32
    wrap = i2 >= NP
    return jnp.where(wrap, b + 1, b), jnp.where(wrap, i2 - NP, i2)

  def issue_inputs(b, i_out, par):
    icl = jnp.maximum(i_out - 1, 0)
    sbase = (b * 16 + jnp.minimum(icl, 15)) * 16
    pltpu.async_copy(ab_hbm.at[b, i_out], ab_v.at[pl.ds(par * ABV, ABP)],
                     sem_in)
    pltpu.async_copy(sp_hbm.at[b, icl], sp_v.at[pl.ds(par * N, N)], sem_in)
    pltpu.async_copy(sel_hbm.at[pl.ds(sbase * D_EDGE, 256)],
                     attr_v.at[pl.ds(par * 256, 256)], sem_in)

  def drain_inputs():
    pltpu.make_async_copy(ab_hbm.at[0, 0], ab_v.at[pl.ds(0, ABP)],
                          sem_in).wait()
    pltpu.make_async_copy(sp_hbm.at[0, 0], sp_v.at[pl.ds(0, N)],
                          sem_in).wait()
    pltpu.make_async_copy(sel_hbm.at[pl.ds(0, 256)], attr_v.at[pl.ds(0, 256)],
                          sem_in).wait()

  def drain_outputs():
    for _ in range(H):
      pltpu.make_async_copy(out_hbm.at[0, 0, 0], outf.at[pl.ds(0, NP)],
                            sem_out).wait()

  issue_inputs(jnp.int32(0), jnp.int32(wid), 0)

  def task_body(k, carry):
    b, i_out = carry
    par = jnp.bitwise_and(k, 1)
    pab = par * ABV
    psp = par * N
    pat = par * 256
    pob = par * OBUF
    is_row0 = i_out == 0
    icl = jnp.maximum(i_out - 1, 0)
    is_edge = jnp.logical_and(jnp.logical_not(is_row0), icl < 16)
    sbase = (b * 16 + jnp.minimum(icl, 15)) * 16

    drain_inputs()
    bn, in_ = advance(b, i_out)

    @pl.when(k + 1 < ntask)
    def _():
      issue_inputs(bn, in_, jnp.bitwise_xor(par, 1))

    @pl.when(k >= 2)
    def _():
      drain_outputs()

    for jg in range(16):
      o = jg * LANES
      tvec[pl.ds(o, LANES)] = 2.0 * plsc.load_gather(ab_v,
                                                     [iota + (pab + o + 1)])
      sp32[pl.ds(o, LANES)] = sp_v[pl.ds(psp + o, LANES)] * 32

    ab0 = ab_v[pl.ds(pab, LANES)][0]

    for hg in range(2):
      hv = iota + hg * LANES
      val = 2.0 * ab0 + virt_v[pl.ds(hg * LANES, LANES)]
      plsc.store_scatter(outf, [pob + hv * OROW], val)

    def jg_body(jg, _):
      o = jg * LANES
      sp32a = sp32[pl.ds(o, LANES)]
      sp32b = sp32[pl.ds(o + LANES, LANES)]
      tva = tvec[pl.ds(o, LANES)]
      tvb = tvec[pl.ds(o + LANES, LANES)]
      base = pob + o + 1
      for h in range(H):
        ga = plsc.load_gather(emb_v, [sp32a + h])
        gb = plsc.load_gather(emb_v, [sp32b + h])
        outf[pl.ds(base + h * OROW, LANES)] = tva + ga
        outf[pl.ds(base + h * OROW + LANES, LANES)] = tvb + gb
      return 0

    lax.fori_loop(0, 8, lambda jg, c: jg_body(jg * 2, c), 0)

    @pl.when(is_row0)
    def _():

      def vh_body(h, _):
        vhv = plsc.load_gather(virt_v, [zeros_i + h])
        base = (pob + h * OROW + 1) + iota
        for jg in range(16):
          o = jg * LANES
          plsc.store_scatter(outf, [base + o], tvec[pl.ds(o, LANES)] + vhv)
        return 0

      lax.fori_loop(0, H, vh_body, 0)

    @pl.when(is_edge)
    def _():
      wmask = wm_v[pl.ds(sbase, LANES)]
      for d in range(D_EDGE):
        attr_t[pl.ds(d * LANES, LANES)] = plsc.load_gather(
            attr_v, [pat + iota * D_EDGE + d])

      def eh_body(h, _):
        acc = jnp.zeros((LANES,), jnp.float32)
        for d in range(D_EDGE):
          wv = plsc.load_gather(wedge_v, [zeros_i + (d * H + h)])
          acc = acc + attr_t[pl.ds(d * LANES, LANES)] * wv
        idxv = (pob + h * OROW + 1) + iota
        cur = plsc.load_gather(outf, [idxv])
        plsc.store_scatter(outf, [idxv], cur + acc * wmask)
        return 0

      lax.fori_loop(0, H, eh_body, 0)

    for h in range(H):
      pltpu.async_copy(outf.at[pl.ds(pob + h * OROW, NP)],
                       out_hbm.at[b, h, i_out], sem_out)

    return (bn, in_)

  lax.fori_loop(0, ntask, task_body, (jnp.int32(0), jnp.int32(wid)))
  drain_outputs()
  drain_outputs()


def _assemble(ab_pad, spatial_pos, emb_flat, wedge_flat, virt, sel_flat,
              w_mask):
  mesh = plsc.VectorSubcoreMesh(core_axis_name="c", subcore_axis_name="s")
  f = pl.kernel(
      _assemble_kernel,
      out_type=jax.ShapeDtypeStruct((B, H, NP, NP), jnp.float32),
      mesh=mesh,
      scratch_types=[
          pltpu.VMEM((2 * ABV,), jnp.float32),
          pltpu.VMEM((2 * N,), jnp.int32),
          pltpu.VMEM((N,), jnp.int32),
          pltpu.VMEM((N,), jnp.float32),
          pltpu.VMEM((2 * 256,), jnp.float32),
          pltpu.VMEM((256,), jnp.float32),
          pltpu.VMEM((2 * OBUF,), jnp.float32),
          pltpu.VMEM((NUM_SPATIAL * H,), jnp.float32),
          pltpu.VMEM((D_EDGE * H,), jnp.float32),
          pltpu.VMEM((H,), jnp.float32),
          pltpu.VMEM((NSLOT,), jnp.float32),
          pltpu.SemaphoreType.DMA,
          pltpu.SemaphoreType.DMA,
      ],
      compiler_params=pltpu.CompilerParams(needs_layout_passes=False,
                                           use_tc_tiling_on_sc=False),
  )
  return f(ab_pad, spatial_pos, emb_flat, wedge_flat, virt, sel_flat, w_mask)


def kernel(x, edge_index, edge_attr, path_index, attn_bias, spatial_pos,
           W_edge, spatial_emb, virt_dist):
  del x, path_index
  ab_pad = jnp.pad(attn_bias, ((0, 0), (0, 0), (0, ABP - NP)))
  sel, w_mask = _edge_select(edge_index, edge_attr)
  out = _assemble(ab_pad, spatial_pos, spatial_emb.reshape(-1),
                  W_edge.reshape(-1), virt_dist.reshape(-1), sel, w_mask)
  return out


# speedup vs baseline: 1.5023x; 1.0352x over previous
import functools

import jax
import jax.numpy as jnp
from jax import lax
from jax.experimental import pallas as pl
from jax.experimental.pallas import tpu as pltpu
from jax.experimental.pallas import tpu_sc as plsc

B, N, H, E, D_EDGE, NUM_SPATIAL = 16, 256, 32, 131072, 16, 512
NSLOT = 4096
LANES = 16
EDGE_CHUNK = 2048


def _edge_select_kernel(ei0_hbm, ei1_hbm, ei2_hbm, ea_hbm, sel_hbm, w_hbm, r0,
                        r1, r2, m2d, w_v, kbuf, idx8_v, gbuf, selflat, sem):
  wid = lax.axis_index("s") * 2 + lax.axis_index("c")

  @pl.when(wid == 0)
  def _():
    iota = lax.iota(jnp.int32, LANES)
    zeros_i = jnp.zeros((LANES,), jnp.int32)
    zeros_f = jnp.zeros((LANES,), jnp.float32)
    ones_f = jnp.ones((LANES,), jnp.float32)

    def init_body(k, _):
      w_v[pl.ds(k * LANES, LANES)] = zeros_f
      row = k >> 3
      col = (k & 7) * LANES
      m2d[row, pl.ds(col, LANES)] = zeros_i
      return 0

    lax.fori_loop(0, NSLOT // LANES, init_body, 0)
    kbuf[pl.ds(LANES, LANES)] = jnp.full((LANES,), 2**30, jnp.int32)

    nsteps = EDGE_CHUNK // LANES

    def issue_chunk(c, par):
      o = par * EDGE_CHUNK
      pltpu.async_copy(ei0_hbm.at[pl.ds(c * EDGE_CHUNK, EDGE_CHUNK)],
                       r0.at[pl.ds(o, EDGE_CHUNK)], sem)
      pltpu.async_copy(ei1_hbm.at[pl.ds(c * EDGE_CHUNK, EDGE_CHUNK)],
                       r1.at[pl.ds(o, EDGE_CHUNK)], sem)
      pltpu.async_copy(ei2_hbm.at[pl.ds(c * EDGE_CHUNK, EDGE_CHUNK)],
                       r2.at[pl.ds(o, EDGE_CHUNK)], sem)

    def drain_chunk():
      for _ in range(3):
        pltpu.make_async_copy(ei0_hbm.at[pl.ds(0, EDGE_CHUNK)],
                              r0.at[pl.ds(0, EDGE_CHUNK)], sem).wait()

    issue_chunk(0, 0)

    def chunk_body(c, _):
      par = jnp.bitwise_and(c, 1)
      drain_chunk()

      @pl.when(c + 1 < E // EDGE_CHUNK)
      def _():
        issue_chunk(c + 1, jnp.bitwise_xor(par, 1))

      def step_body(s, _):
        off = par * EDGE_CHUNK + s * LANES
        ia = r0[pl.ds(off, LANES)]
        ib = r1[pl.ds(off, LANES)]
        ic = r2[pl.ds(off, LANES)]
        slot = ia * 256 + ib * 16 + ic
        e = (c * EDGE_CHUNK + s * LANES) + iota
        key = slot * (2**17) + e
        sk, _unused = plsc.sort_key_val(key, e)
        kbuf[pl.ds(0, LANES)] = sk
        nxt = plsc.load_gather(kbuf, [iota + 1])
        slot_s = lax.shift_right_arithmetic(sk, 17)
        keep = jnp.not_equal(slot_s, lax.shift_right_arithmetic(nxt, 17))
        es = jnp.bitwise_and(sk, 2**17 - 1)
        row = lax.shift_right_arithmetic(slot_s, 7)
        col = jnp.bitwise_and(slot_s, 127)
        plsc.store_scatter(m2d, [row, col], es, mask=keep)
        plsc.store_scatter(w_v, [slot_s], ones_f, mask=keep)
        return 0

      lax.fori_loop(0, nsteps, step_body, 0)
      return 0

    lax.fori_loop(0, E // EDGE_CHUNK, chunk_body, 0)

    def chunk_gather(r, _):

      def idx_body(g, _):
        mv = plsc.load_gather(m2d, [zeros_i + r, g * LANES + iota])
        idx8_v[pl.ds(g * LANES, LANES)] = lax.shift_right_arithmetic(mv, 3)
        return 0

      lax.fori_loop(0, 8, idx_body, 0)
      pltpu.async_copy(ea_hbm.at[idx8_v], gbuf, sem).wait()

      def ext_body(k, _):
        mkv = plsc.load_gather(m2d, [zeros_i + r, zeros_i + k])
        sub = jnp.bitwise_and(mkv, 7) * D_EDGE
        val = plsc.load_gather(gbuf, [zeros_i + k, sub + iota])
        selflat[pl.ds(k * D_EDGE, LANES)] = val
        return 0

      lax.fori_loop(0, 128, ext_body, 0)
      pltpu.sync_copy(selflat,
                      sel_hbm.at[pl.ds(r * 128 * D_EDGE, 128 * D_EDGE)])
      return 0

    lax.fori_loop(0, NSLOT // 128, chunk_gather, 0)
    pltpu.sync_copy(w_v, w_hbm)


def _edge_select(edge_index, edge_attr):
  mesh = plsc.VectorSubcoreMesh(core_axis_name="c", subcore_axis_name="s")
  f = pl.kernel(
      _edge_select_kernel,
      out_type=(jax.ShapeDtypeStruct((NSLOT * D_EDGE,), jnp.float32),
                jax.ShapeDtypeStruct((NSLOT,), jnp.float32)),
      mesh=mesh,
      scratch_types=[
          pltpu.VMEM((2 * EDGE_CHUNK,), jnp.int32),
          pltpu.VMEM((2 * EDGE_CHUNK,), jnp.int32),
          pltpu.VMEM((2 * EDGE_CHUNK,), jnp.int32),
          pltpu.VMEM((32, 128), jnp.int32),
          pltpu.VMEM((NSLOT,), jnp.float32),
          pltpu.VMEM((2 * LANES,), jnp.int32),
          pltpu.VMEM((128,), jnp.int32),
          pltpu.VMEM((128, 128), jnp.float32),
          pltpu.VMEM((128 * D_EDGE,), jnp.float32),
          pltpu.SemaphoreType.DMA,
      ],
      compiler_params=pltpu.CompilerParams(needs_layout_passes=False),
  )
  return f(edge_index[0], edge_index[1], edge_index[2],
           edge_attr.reshape(E // 8, 128))


NP = N + 1
ABP = 264
OROW = 272
OBUF = H * OROW


ABV = 272


def _assemble_kernel(ab_hbm, sp_hbm, emb_hbm, wedge_hbm, virt_hbm, sel_hbm,
                     wm_hbm, out_hbm, ab_v, sp_v, sp32, tvec, attr_v, attr_t,
                     outf, emb_v, wedge_v, virt_v, wm_v, sem_in, sem_out):
  wid = lax.axis_index("s") * 2 + lax.axis_index("c")
  iota = lax.iota(jnp.int32, LANES)
  zeros_i = jnp.zeros((LANES,), jnp.int32)

  pltpu.sync_copy(emb_hbm, emb_v)
  pltpu.sync_copy(wedge_hbm, wedge_v)
  pltpu.sync_copy(virt_hbm, virt_v)
  pltpu.sync_copy(wm_hbm, wm_v)

  ntask = jnp.where(wid < LANES, 129, 128)

  def advance(b, i_out):
    i2 = i_out + 32
    wrap = i2 >= NP
    return jnp.where(wrap, b + 1, b), jnp.where(wrap, i2 - NP, i2)

  def issue_inputs(b, i_out, par):
    icl = jnp.maximum(i_out - 1, 0)
    sbase = (b * 16 + jnp.minimum(icl, 15)) * 16
    pltpu.async_copy(ab_hbm.at[b, i_out], ab_v.at[pl.ds(par * ABV, ABP)],
                     sem_in)
    pltpu.async_copy(sp_hbm.at[b, icl], sp_v.at[pl.ds(par * N, N)], sem_in)
    pltpu.async_copy(sel_hbm.at[pl.ds(sbase * D_EDGE, 256)],
                     attr_v.at[pl.ds(par * 256, 256)], sem_in)

  def drain_inputs():
    pltpu.make_async_copy(ab_hbm.at[0, 0], ab_v.at[pl.ds(0, ABP)],
                          sem_in).wait()
    pltpu.make_async_copy(sp_hbm.at[0, 0], sp_v.at[pl.ds(0, N)],
                          sem_in).wait()
    pltpu.make_async_copy(sel_hbm.at[pl.ds(0, 256)], attr_v.at[pl.ds(0, 256)],
                          sem_in).wait()

  def drain_outputs():
    for _ in range(H):
      pltpu.make_async_copy(out_hbm.at[0, 0, 0], outf.at[pl.ds(0, NP)],
                            sem_out).wait()

  issue_inputs(jnp.int32(0), jnp.int32(wid), 0)

  def task_body(k, carry):
    b, i_out = carry
    par = jnp.bitwise_and(k, 1)
    pab = par * ABV
    psp = par * N
    pat = par * 256
    pob = par * OBUF
    is_row0 = i_out == 0
    icl = jnp.maximum(i_out - 1, 0)
    is_edge = jnp.logical_and(jnp.logical_not(is_row0), icl < 16)
    sbase = (b * 16 + jnp.minimum(icl, 15)) * 16

    drain_inputs()
    bn, in_ = advance(b, i_out)

    @pl.when(k + 1 < ntask)
    def _():
      issue_inputs(bn, in_, jnp.bitwise_xor(par, 1))

    @pl.when(k >= 2)
    def _():
      drain_outputs()

    for jg in range(16):
      o = jg * LANES
      tvec[pl.ds(o, LANES)] = 2.0 * plsc.load_gather(ab_v,
                                                     [iota + (pab + o + 1)])
      sp32[pl.ds(o, LANES)] = sp_v[pl.ds(psp + o, LANES)] * 32

    ab0 = ab_v[pl.ds(pab, LANES)][0]

    for hg in range(2):
      hv = iota + hg * LANES
      val = 2.0 * ab0 + virt_v[pl.ds(hg * LANES, LANES)]
      plsc.store_scatter(outf, [pob + hv * OROW], val)

    def jg_body(jg, _):
      o = jg * LANES
      sps = [sp32[pl.ds(o + u * LANES, LANES)] for u in range(4)]
      tvs = [tvec[pl.ds(o + u * LANES, LANES)] for u in range(4)]
      base = pob + o + 1
      for h in range(H):
        gs = [plsc.load_gather(emb_v, [sp + h]) for sp in sps]
        for u in range(4):
          outf[pl.ds(base + h * OROW + u * LANES, LANES)] = tvs[u] + gs[u]
      return 0

    lax.fori_loop(0, 4, lambda jg, c: jg_body(jg * 4, c), 0)

    @pl.when(is_row0)
    def _():

      def vh_body(h, _):
        vhv = plsc.load_gather(virt_v, [zeros_i + h])
        base = (pob + h * OROW + 1) + iota
        for jg in range(16):
          o = jg * LANES
          plsc.store_scatter(outf, [base + o], tvec[pl.ds(o, LANES)] + vhv)
        return 0

      lax.fori_loop(0, H, vh_body, 0)

    @pl.when(is_edge)
    def _():
      wmask = wm_v[pl.ds(sbase, LANES)]
      for d in range(D_EDGE):
        attr_t[pl.ds(d * LANES, LANES)] = plsc.load_gather(
            attr_v, [pat + iota * D_EDGE + d])

      def eh_body(h, _):
        acc = jnp.zeros((LANES,), jnp.float32)
        for d in range(D_EDGE):
          wv = plsc.load_gather(wedge_v, [zeros_i + (d * H + h)])
          acc = acc + attr_t[pl.ds(d * LANES, LANES)] * wv
        idxv = (pob + h * OROW + 1) + iota
        cur = plsc.load_gather(outf, [idxv])
        plsc.store_scatter(outf, [idxv], cur + acc * wmask)
        return 0

      lax.fori_loop(0, H, eh_body, 0)

    for h in range(H):
      pltpu.async_copy(outf.at[pl.ds(pob + h * OROW, NP)],
                       out_hbm.at[b, h, i_out], sem_out)

    return (bn, in_)

  lax.fori_loop(0, ntask, task_body, (jnp.int32(0), jnp.int32(wid)))
  drain_outputs()
  drain_outputs()


def _assemble(ab_pad, spatial_pos, emb_flat, wedge_flat, virt, sel_flat,
              w_mask):
  mesh = plsc.VectorSubcoreMesh(core_axis_name="c", subcore_axis_name="s")
  f = pl.kernel(
      _assemble_kernel,
      out_type=jax.ShapeDtypeStruct((B, H, NP, NP), jnp.float32),
      mesh=mesh,
      scratch_types=[
          pltpu.VMEM((2 * ABV,), jnp.float32),
          pltpu.VMEM((2 * N,), jnp.int32),
          pltpu.VMEM((N,), jnp.int32),
          pltpu.VMEM((N,), jnp.float32),
          pltpu.VMEM((2 * 256,), jnp.float32),
          pltpu.VMEM((256,), jnp.float32),
          pltpu.VMEM((2 * OBUF,), jnp.float32),
          pltpu.VMEM((NUM_SPATIAL * H,), jnp.float32),
          pltpu.VMEM((D_EDGE * H,), jnp.float32),
          pltpu.VMEM((H,), jnp.float32),
          pltpu.VMEM((NSLOT,), jnp.float32),
          pltpu.SemaphoreType.DMA,
          pltpu.SemaphoreType.DMA,
      ],
      compiler_params=pltpu.CompilerParams(needs_layout_passes=False,
                                           use_tc_tiling_on_sc=False),
  )
  return f(ab_pad, spatial_pos, emb_flat, wedge_flat, virt, sel_flat, w_mask)


def kernel(x, edge_index, edge_attr, path_index, attn_bias, spatial_pos,
           W_edge, spatial_emb, virt_dist):
  del x, path_index
  ab_pad = jnp.pad(attn_bias, ((0, 0), (0, 0), (0, ABP - NP)))
  sel, w_mask = _edge_select(edge_index, edge_attr)
  out = _assemble(ab_pad, spatial_pos, spatial_emb.reshape(-1),
                  W_edge.reshape(-1), virt_dist.reshape(-1), sel, w_mask)
  return out
